# baseline (speedup 1.0000x reference)
import functools

import jax
import jax.numpy as jnp
from jax import lax
from jax.experimental import pallas as pl
from jax.experimental.pallas import tpu as pltpu
from jax.experimental.pallas import tpu_sc as plsc

_D = 64
_B = 4096 * 200
_NC, _NS, _L = 2, 16, 16
_NW = _NC * _NS
_BPW = _B // _NW
_CHUNK = 800
_NCHUNKS = _BPW // _CHUNK
_SCALE = 8.0


def _emb_body(idx_hbm, table_hbm, out_hbm, idx_v, rows_v, sem):
    wid = lax.axis_index("s") * _NC + lax.axis_index("c")
    base = wid * _BPW
    pltpu.sync_copy(idx_hbm.at[pl.ds(base, _BPW)], idx_v)

    def chunk_body(c, carry):
        off = c * _CHUNK
        pltpu.async_copy(
            table_hbm.at[idx_v.at[pl.ds(off, _CHUNK)]], rows_v, sem
        ).wait()

        def scale_body(i, carry2):
            for j in range(_D // _L):
                sl = pl.ds(j * _L, _L)
                rows_v[i, sl] = rows_v[i, sl] * _SCALE
            return carry2

        lax.fori_loop(0, _CHUNK, scale_body, 0, unroll=4)
        pltpu.sync_copy(rows_v, out_hbm.at[pl.ds(base + off, _CHUNK)])
        return carry

    lax.fori_loop(0, _NCHUNKS, chunk_body, 0)


@jax.jit
def _emb(idx_flat, table):
    mesh = plsc.VectorSubcoreMesh(
        core_axis_name="c", subcore_axis_name="s",
        num_cores=_NC, num_subcores=_NS,
    )
    f = pl.kernel(
        _emb_body,
        out_type=jax.ShapeDtypeStruct((_B, _D), jnp.float32),
        mesh=mesh,
        scratch_types=[
            pltpu.VMEM((_BPW,), jnp.int32),
            pltpu.VMEM((_CHUNK, _D), jnp.float32),
            pltpu.SemaphoreType.DMA,
        ],
        compiler_params=pltpu.CompilerParams(use_tc_tiling_on_sc=False),
    )
    return f(idx_flat, table)


def kernel(inputs, table):
    flat = inputs.reshape(-1)
    out = _emb(flat, table)
    return out.reshape(inputs.shape + (_D,))


# speedup vs baseline: 1.0553x; 1.0553x over previous
import functools

import jax
import jax.numpy as jnp
from jax import lax
from jax.experimental import pallas as pl
from jax.experimental.pallas import tpu as pltpu
from jax.experimental.pallas import tpu_sc as plsc

_D = 64
_B = 4096 * 200
_NC, _NS, _L = 2, 16, 16
_NW = _NC * _NS
_BPW = _B // _NW
_CHUNK = 800
_NCHUNKS = _BPW // _CHUNK
_SCALE = 8.0


def _emb_body(idx_hbm, table_hbm, out_hbm, idx_v, rows0, rows1,
              gsem0, gsem1, osem0, osem1):
    rows = (rows0, rows1)
    gsems = (gsem0, gsem1)
    osems = (osem0, osem1)
    wid = lax.axis_index("s") * _NC + lax.axis_index("c")
    base = wid * _BPW
    pltpu.sync_copy(idx_hbm.at[pl.ds(base, _BPW)], idx_v)

    def start_gather(c, b):
        pltpu.async_copy(
            table_hbm.at[idx_v.at[pl.ds(c * _CHUNK, _CHUNK)]], rows[b],
            gsems[b])

    def scale(b):
        def scale_body(i, carry):
            for j in range(_D // _L):
                sl = pl.ds(j * _L, _L)
                rows[b][i, sl] = rows[b][i, sl] * _SCALE
            return carry
        lax.fori_loop(0, _CHUNK, scale_body, 0, unroll=4)

    start_gather(0, 0)

    def pair_body(g, carry):
        for b in range(2):
            c = 2 * g + b
            nb = 1 - b
            pltpu.make_async_copy(
                table_hbm.at[idx_v.at[pl.ds(c * _CHUNK, _CHUNK)]], rows[b],
                gsems[b]).wait()
            @pl.when(c >= 1)
            def _():
                pltpu.make_async_copy(
                    rows[nb], out_hbm.at[pl.ds(base, _CHUNK)],
                    osems[nb]).wait()

            @pl.when(c + 1 < _NCHUNKS)
            def _():
                start_gather(c + 1, nb)

            scale(b)
            pltpu.async_copy(
                rows[b], out_hbm.at[pl.ds(base + c * _CHUNK, _CHUNK)],
                osems[b])
        return carry

    lax.fori_loop(0, _NCHUNKS // 2, pair_body, 0)
    lb = (_NCHUNKS - 1) % 2
    pltpu.make_async_copy(
        rows[lb], out_hbm.at[pl.ds(base, _CHUNK)], osems[lb]).wait()


@jax.jit
def _emb(idx_flat, table):
    mesh = plsc.VectorSubcoreMesh(
        core_axis_name="c", subcore_axis_name="s",
        num_cores=_NC, num_subcores=_NS,
    )
    f = pl.kernel(
        _emb_body,
        out_type=jax.ShapeDtypeStruct((_B, _D), jnp.float32),
        mesh=mesh,
        scratch_types=[
            pltpu.VMEM((_BPW,), jnp.int32),
            pltpu.VMEM((_CHUNK, _D), jnp.float32),
            pltpu.VMEM((_CHUNK, _D), jnp.float32),
            pltpu.SemaphoreType.DMA,
            pltpu.SemaphoreType.DMA,
            pltpu.SemaphoreType.DMA,
            pltpu.SemaphoreType.DMA,
        ],
        compiler_params=pltpu.CompilerParams(use_tc_tiling_on_sc=False),
    )
    return f(idx_flat, table)


def kernel(inputs, table):
    flat = inputs.reshape(-1)
    out = _emb(flat, table)
    return out.reshape(inputs.shape + (_D,))
